# baseline (speedup 1.0000x reference)
import functools

import jax
import jax.numpy as jnp
from jax import lax
from jax.experimental import pallas as pl
from jax.experimental.pallas import tpu as pltpu
from jax.experimental.pallas import tpu_sc as plsc

_N, _D = 10000, 128
_E = 320000
_NC, _NS = 2, 16
_NW = _NC * _NS
_R = 320
_NPAD = _NW * _R
_C = 5000
_NCHUNK = _E // _C
_G = 64
_W = 3
_WG = _W * _G
_CAP = 6592
_MOVE = 1024
_CATCH = 512
_NEG_INF = float("-inf")


def _sc_segment_max(x, src, dst):
  mesh = plsc.VectorSubcoreMesh(core_axis_name="c", subcore_axis_name="s")

  @functools.partial(
      pl.kernel,
      out_type=jax.ShapeDtypeStruct((_NPAD, _D), jnp.float32),
      mesh=mesh,
      compiler_params=pltpu.CompilerParams(needs_layout_passes=False),
      scratch_types=[
          pltpu.VMEM((_R + 1, _D), jnp.float32),
          pltpu.VMEM((2 * _C,), jnp.int32),
          pltpu.VMEM((2 * _C,), jnp.int32),
          pltpu.VMEM((_CAP,), jnp.int32),
          pltpu.VMEM((_CAP,), jnp.int32),
          pltpu.VMEM((_WG, _D), jnp.float32),
          pltpu.VMEM((_WG,), jnp.int32),
          pltpu.VMEM((_WG,), jnp.int32),
          pltpu.VMEM((_G, _D), jnp.float32),
          pltpu.VMEM((_G,), jnp.int32),
          pltpu.SemaphoreType.DMA,
          pltpu.SemaphoreType.DMA,
          pltpu.SemaphoreType.DMA,
          pltpu.SemaphoreType.DMA,
          pltpu.SemaphoreType.DMA,
          pltpu.SemaphoreType.DMA,
      ],
  )
  def kern(x_hbm, src_hbm, dst_hbm, out_hbm,
           acc, sbuf, dbuf, srcc, dstc, rows, idxb, dsnap, rowsc, idxc,
           sema, semb, gs0, gs1, gs2, csem):
    wid = lax.axis_index("s") * _NC + lax.axis_index("c")
    lo = wid * _R
    gsems = [gs0, gs1, gs2]

    neg = jnp.full((16,), _NEG_INF, jnp.float32)
    zero16 = jnp.zeros((16,), jnp.int32)
    dump16 = jnp.full((16,), _R, jnp.int32)

    @pl.loop(0, _R + 1)
    def _(i):
      for j in range(_D // 16):
        acc[i, pl.ds(j * 16, 16)] = neg

    def start_chunk_dma(ci, par, sem):
      sl = pl.ds(par * _C, _C)
      pltpu.async_copy(src_hbm.at[pl.ds(ci * _C, _C)], sbuf.at[sl], sem)
      pltpu.async_copy(dst_hbm.at[pl.ds(ci * _C, _C)], dbuf.at[sl], sem)

    def wait_chunk_dma(par, sem):
      sl = pl.ds(par * _C, _C)
      pltpu.make_async_copy(src_hbm.at[pl.ds(0, _C)], sbuf.at[sl], sem).wait()
      pltpu.make_async_copy(dst_hbm.at[pl.ds(0, _C)], dbuf.at[sl], sem).wait()

    def issue_window(w):
      ws = pl.ds(w * _G, _G)
      for j in range(_G // 16):
        idxb[pl.ds(w * _G + j * 16, 16)] = srcc[pl.ds(w * _G + j * 16, 16)]
        dsnap[pl.ds(w * _G + j * 16, 16)] = dstc[pl.ds(w * _G + j * 16, 16)]
      pltpu.async_copy(x_hbm.at[idxb.at[ws]], rows.at[ws], gsems[w])

    def max_rows(dl, row_ref, row):
      avs = [acc[dl, pl.ds(j * 16, 16)] for j in range(_D // 16)]
      rvs = [row_ref[row, pl.ds(j * 16, 16)] for j in range(_D // 16)]
      for j in range(_D // 16):
        acc[dl, pl.ds(j * 16, 16)] = jnp.maximum(avs[j], rvs[j])

    def accum_window(w):
      ws = pl.ds(w * _G, _G)
      pltpu.make_async_copy(x_hbm.at[idxb.at[ws]], rows.at[ws],
                            gsems[w]).wait()

      @pl.loop(0, _G // 16)
      def _(rg):
        dv = dsnap[pl.ds(w * _G + rg * 16, 16)]
        for r in range(16):
          max_rows(dv[r], rows, w * _G + rg * 16 + r)

    def catch_drain(t):
      for j in range(_G // 16):
        idxc[pl.ds(j * 16, 16)] = srcc[pl.ds(t + j * 16, 16)]
      pltpu.async_copy(x_hbm.at[idxc], rowsc, csem).wait()

      @pl.loop(0, _G // 16)
      def _(rg):
        dv = dstc[pl.ds(t + rg * 16, 16)]
        for r in range(16):
          max_rows(dv[r], rowsc, rg * 16 + r)

    def do_chunk(par, cnt0):
      base = par * _C

      @pl.loop(0, _C // 16, init_carry=cnt0)
      def cnt(g, c):
        d = dbuf[pl.ds(base + g * 16, 16)]
        s = sbuf[pl.ds(base + g * 16, 16)]
        msk = (d >= lo) & (d < lo + _R)
        plsc.store_compressed(srcc.at[pl.ds(c, 16)], s, mask=msk)
        plsc.store_compressed(dstc.at[pl.ds(c, 16)], d - lo, mask=msk)
        return c + jnp.sum(msk.astype(jnp.int32))

      for j in range(_WG // 16):
        srcc[pl.ds(cnt + j * 16, 16)] = zero16
        dstc[pl.ds(cnt + j * 16, 16)] = dump16

      for w in range(_W):
        accum_window(w)
        issue_window(w)

      t = jnp.minimum(jnp.int32(_WG), cnt)

      def catch_cond(st):
        c2, t2 = st
        return c2 - t2 > _CATCH

      def catch_body(st):
        c2, t2 = st
        catch_drain(t2)
        return (c2, t2 + _G)

      cnt, t = lax.while_loop(catch_cond, catch_body, (cnt, t))

      @pl.loop(0, _MOVE // 16)
      def _(mi):
        srcc[pl.ds(mi * 16, 16)] = srcc[pl.ds(t + mi * 16, 16)]
        dstc[pl.ds(mi * 16, 16)] = dstc[pl.ds(t + mi * 16, 16)]

      return cnt - t

    for j in range(_WG // 16):
      idxb[pl.ds(j * 16, 16)] = zero16
      dsnap[pl.ds(j * 16, 16)] = dump16
    for w in range(_W):
      ws = pl.ds(w * _G, _G)
      pltpu.async_copy(x_hbm.at[idxb.at[ws]], rows.at[ws], gsems[w])

    start_chunk_dma(0, 0, sema)

    @pl.loop(0, _NCHUNK // 2, init_carry=jnp.int32(0))
    def fcnt(i, cnt):
      ci = i * 2
      wait_chunk_dma(0, sema)
      start_chunk_dma(ci + 1, 1, semb)
      cnt = do_chunk(0, cnt)
      wait_chunk_dma(1, semb)

      @pl.when(ci + 2 < _NCHUNK)
      def _():
        start_chunk_dma(ci + 2, 0, sema)

      cnt = do_chunk(1, cnt)
      return cnt

    for w in range(_W):
      accum_window(w)

    for j in range(_G // 16):
      srcc[pl.ds(fcnt + j * 16, 16)] = zero16
      dstc[pl.ds(fcnt + j * 16, 16)] = dump16

    @pl.loop(0, _CATCH // _G, init_carry=jnp.int32(0))
    def _(k, t):
      catch_drain(t)
      return jnp.minimum(t + _G, fcnt)

    pltpu.sync_copy(acc.at[pl.ds(0, _R)], out_hbm.at[pl.ds(lo, _R)])

  return kern(x, src, dst)


def _tc_linear(x, m, W, b):
  br = 400
  nb = _N // br

  def body(x_ref, m_ref, w_ref, b_ref, o_ref):
    xv = x_ref[...]
    mv = m_ref[...]
    md = jnp.where(mv == _NEG_INF, jnp.float32(0), mv - xv)
    o_ref[...] = (
        jnp.dot(xv, w_ref[0:_D, :], preferred_element_type=jnp.float32)
        + jnp.dot(md, w_ref[_D:2 * _D, :], preferred_element_type=jnp.float32)
        + b_ref[...]
    )

  return pl.pallas_call(
      body,
      grid=(nb,),
      in_specs=[
          pl.BlockSpec((br, _D), lambda i: (i, 0)),
          pl.BlockSpec((br, _D), lambda i: (i, 0)),
          pl.BlockSpec((2 * _D, _D), lambda i: (0, 0)),
          pl.BlockSpec((1, _D), lambda i: (0, 0)),
      ],
      out_specs=pl.BlockSpec((br, _D), lambda i: (i, 0)),
      out_shape=jax.ShapeDtypeStruct((_N, _D), jnp.float32),
  )(x, m, W, b.reshape(1, _D))


def kernel(x, edge_index, W, b):
  src = edge_index[0]
  dst = edge_index[1]
  m = _sc_segment_max(x, src, dst)
  return _tc_linear(x, m, W, b)


# speedup vs baseline: 2.6268x; 2.6268x over previous
import functools

import jax
import jax.numpy as jnp
from jax import lax
from jax.experimental import pallas as pl
from jax.experimental.pallas import tpu as pltpu
from jax.experimental.pallas import tpu_sc as plsc

_N, _D = 10000, 128
_E = 320000
_NC, _NS = 2, 16
_NW = _NC * _NS
_R = 320
_NPAD = _NW * _R
_C = 4000
_NCHUNK = _E // _C
_G = 128
_NFIN = 4
_NBLK = _NCHUNK + _NFIN
_CAP = 5568
_MOVE = 1024
_CATCH = 512
_NEG_INF = float("-inf")


def _sc_segment_max(x, src, dst):
  mesh = plsc.VectorSubcoreMesh(core_axis_name="c", subcore_axis_name="s")

  @functools.partial(
      pl.kernel,
      out_type=(
          jax.ShapeDtypeStruct((_NPAD, _D), jnp.float32),
          jax.ShapeDtypeStruct((_NW * _NBLK * _G,), jnp.int32),
          jax.ShapeDtypeStruct((_NW * _NBLK * _G,), jnp.int32),
      ),
      mesh=mesh,
      compiler_params=pltpu.CompilerParams(needs_layout_passes=False),
      scratch_types=[
          pltpu.VMEM((_R + 1, _D), jnp.float32),
          pltpu.VMEM((2 * _C,), jnp.int32),
          pltpu.VMEM((2 * _C,), jnp.int32),
          pltpu.VMEM((_CAP,), jnp.int32),
          pltpu.VMEM((_CAP,), jnp.int32),
          pltpu.VMEM((_G, _D), jnp.float32),
          pltpu.VMEM((_G,), jnp.int32),
          pltpu.SemaphoreType.DMA,
          pltpu.SemaphoreType.DMA,
          pltpu.SemaphoreType.DMA,
          pltpu.SemaphoreType.DMA,
      ],
  )
  def kern(x_hbm, src_hbm, dst_hbm, out_hbm, lsrc_hbm, ldst_hbm,
           acc, sbuf, dbuf, srcc, dstc, rowsc, idxc,
           sema, semb, fsem, csem):
    wid = lax.axis_index("s") * _NC + lax.axis_index("c")
    lo = wid * _R
    lbase = wid * (_NBLK * _G)

    neg = jnp.full((16,), _NEG_INF, jnp.float32)
    zero16 = jnp.zeros((16,), jnp.int32)
    dump16 = jnp.full((16,), _R, jnp.int32)

    @pl.loop(0, _R + 1)
    def _(i):
      for j in range(_D // 16):
        acc[i, pl.ds(j * 16, 16)] = neg

    def start_chunk_dma(ci, par, sem):
      sl = pl.ds(par * _C, _C)
      pltpu.async_copy(src_hbm.at[pl.ds(ci * _C, _C)], sbuf.at[sl], sem)
      pltpu.async_copy(dst_hbm.at[pl.ds(ci * _C, _C)], dbuf.at[sl], sem)

    def wait_chunk_dma(par, sem):
      sl = pl.ds(par * _C, _C)
      pltpu.make_async_copy(src_hbm.at[pl.ds(0, _C)], sbuf.at[sl], sem).wait()
      pltpu.make_async_copy(dst_hbm.at[pl.ds(0, _C)], dbuf.at[sl], sem).wait()

    def start_flush(k, bi):
      qs = pl.ds(k * _G, _G)
      dsl = pl.ds(lbase + bi * _G, _G)
      pltpu.async_copy(srcc.at[qs], lsrc_hbm.at[dsl], fsem)
      pltpu.async_copy(dstc.at[qs], ldst_hbm.at[dsl], fsem)

    def wait_flush(k):
      qs = pl.ds(k * _G, _G)
      dsl = pl.ds(lbase, _G)
      pltpu.make_async_copy(srcc.at[qs], lsrc_hbm.at[dsl], fsem).wait()
      pltpu.make_async_copy(dstc.at[qs], ldst_hbm.at[dsl], fsem).wait()

    def max_rows(dl, row_ref, row):
      avs = [acc[dl, pl.ds(j * 16, 16)] for j in range(_D // 16)]
      rvs = [row_ref[row, pl.ds(j * 16, 16)] for j in range(_D // 16)]
      for j in range(_D // 16):
        acc[dl, pl.ds(j * 16, 16)] = jnp.maximum(avs[j], rvs[j])

    def catch_drain(t):
      for j in range(_G // 16):
        idxc[pl.ds(j * 16, 16)] = srcc[pl.ds(t + j * 16, 16)]
      pltpu.async_copy(x_hbm.at[idxc], rowsc, csem).wait()

      @pl.loop(0, _G // 16)
      def _(rg):
        dv = dstc[pl.ds(t + rg * 16, 16)]
        for r in range(16):
          max_rows(dv[r], rowsc, rg * 16 + r)

    def do_chunk(ci, par, cnt0):
      base = par * _C

      @pl.loop(0, _C // 16, init_carry=cnt0)
      def cnt(g, c):
        d = dbuf[pl.ds(base + g * 16, 16)]
        s = sbuf[pl.ds(base + g * 16, 16)]
        msk = (d >= lo) & (d < lo + _R)
        plsc.store_compressed(srcc.at[pl.ds(c, 16)], s, mask=msk)
        plsc.store_compressed(dstc.at[pl.ds(c, 16)], d - lo, mask=msk)
        return c + jnp.sum(msk.astype(jnp.int32))

      for j in range(_G // 16):
        srcc[pl.ds(cnt + j * 16, 16)] = zero16
        dstc[pl.ds(cnt + j * 16, 16)] = dump16

      start_flush(0, ci)
      t = jnp.minimum(jnp.int32(_G), cnt)

      def catch_cond(st):
        c2, t2 = st
        return c2 - t2 > _CATCH

      def catch_body(st):
        c2, t2 = st
        catch_drain(t2)
        return (c2, t2 + _G)

      cnt, t = lax.while_loop(catch_cond, catch_body, (cnt, t))
      wait_flush(0)

      @pl.loop(0, _MOVE // 16)
      def _(mi):
        srcc[pl.ds(mi * 16, 16)] = srcc[pl.ds(t + mi * 16, 16)]
        dstc[pl.ds(mi * 16, 16)] = dstc[pl.ds(t + mi * 16, 16)]

      return cnt - t

    start_chunk_dma(0, 0, sema)

    @pl.loop(0, _NCHUNK // 2, init_carry=jnp.int32(0))
    def fcnt(i, cnt):
      ci = i * 2
      wait_chunk_dma(0, sema)
      start_chunk_dma(ci + 1, 1, semb)
      cnt = do_chunk(ci, 0, cnt)
      wait_chunk_dma(1, semb)

      @pl.when(ci + 2 < _NCHUNK)
      def _():
        start_chunk_dma(ci + 2, 0, sema)

      cnt = do_chunk(ci + 1, 1, cnt)
      return cnt

    for j in range(_NFIN * _G // 16):
      srcc[pl.ds(fcnt + j * 16, 16)] = zero16
      dstc[pl.ds(fcnt + j * 16, 16)] = dump16
    for k in range(_NFIN):
      start_flush(k, _NCHUNK + k)
    for k in range(_NFIN):
      wait_flush(k)

    def p2_body(isrc_vmem, idst_vmem):
      pltpu.sync_copy(x_hbm.at[isrc_vmem], rowsc)

      @pl.loop(0, _G // 16)
      def _(rg):
        dv = idst_vmem[pl.ds(rg * 16, 16)]
        for r in range(16):
          max_rows(dv[r], rowsc, rg * 16 + r)

    pltpu.emit_pipeline(
        p2_body,
        grid=(_NBLK,),
        in_specs=[
            pl.BlockSpec((_G,), index_map=lambda i: (wid * _NBLK + i,)),
            pl.BlockSpec((_G,), index_map=lambda i: (wid * _NBLK + i,)),
        ],
        out_specs=[],
    )(lsrc_hbm, ldst_hbm)

    pltpu.sync_copy(acc.at[pl.ds(0, _R)], out_hbm.at[pl.ds(lo, _R)])

  m, _, _ = kern(x, src, dst)
  return m


def _tc_linear(x, m, W, b):
  br = 400
  nb = _N // br

  def body(x_ref, m_ref, w_ref, b_ref, o_ref):
    xv = x_ref[...]
    mv = m_ref[...]
    md = jnp.where(mv == _NEG_INF, jnp.float32(0), mv - xv)
    o_ref[...] = (
        jnp.dot(xv, w_ref[0:_D, :], preferred_element_type=jnp.float32)
        + jnp.dot(md, w_ref[_D:2 * _D, :], preferred_element_type=jnp.float32)
        + b_ref[...]
    )

  return pl.pallas_call(
      body,
      grid=(nb,),
      in_specs=[
          pl.BlockSpec((br, _D), lambda i: (i, 0)),
          pl.BlockSpec((br, _D), lambda i: (i, 0)),
          pl.BlockSpec((2 * _D, _D), lambda i: (0, 0)),
          pl.BlockSpec((1, _D), lambda i: (0, 0)),
      ],
      out_specs=pl.BlockSpec((br, _D), lambda i: (i, 0)),
      out_shape=jax.ShapeDtypeStruct((_N, _D), jnp.float32),
  )(x, m, W, b.reshape(1, _D))


def kernel(x, edge_index, W, b):
  src = edge_index[0]
  dst = edge_index[1]
  m = _sc_segment_max(x, src, dst)
  return _tc_linear(x, m, W, b)
